# baseline (speedup 1.0000x reference)
import functools

import jax
import jax.numpy as jnp
from jax import lax
from jax.experimental import pallas as pl
from jax.experimental.pallas import tpu as pltpu
from jax.experimental.pallas import tpu_sc as plsc

_B, _L, _D = 16384, 200, 64
_ROW = _D * _L
_OFFS = tuple(range(0, 192, 16)) + (184,)
_XCHUNK = 8
_NOB = 4


@functools.lru_cache(maxsize=1)
def _build():
    info = plsc.get_sparse_core_info()
    nw = info.num_cores * info.num_subcores
    rows_per_w = _B // nw
    n_chunks = rows_per_w // _XCHUNK

    mesh = plsc.VectorSubcoreMesh(core_axis_name="c", subcore_axis_name="s")

    @functools.partial(
        pl.kernel,
        out_type=jax.ShapeDtypeStruct((_B, _D, _L), jnp.float32),
        mesh=mesh,
        scratch_types=[
            pltpu.VMEM((_D * 16,), jnp.float32),
            pltpu.VMEM((_D * 16,), jnp.float32),
            pltpu.VMEM((2, _XCHUNK, _L), jnp.int32),
            pltpu.VMEM((_NOB, _D, _L), jnp.float32),
            pltpu.SemaphoreType.DMA,
            pltpu.SemaphoreType.DMA,
            [pltpu.SemaphoreType.DMA] * _NOB,
        ],
    )
    def k(x_hbm, r0s_hbm, drs_hbm, out_hbm,
          r0s_v, drs_v, xc_v, obuf_v, sx0, sx1, osems):
        c = lax.axis_index("c")
        s = lax.axis_index("s")
        wid = s * info.num_cores + c
        base = wid * rows_per_w
        xsems = (sx0, sx1)

        pltpu.sync_copy(r0s_hbm, r0s_v)
        pltpu.sync_copy(drs_hbm, drs_v)

        def x_slice(ci):
            return x_hbm.at[pl.ds(base + ci * _XCHUNK, _XCHUNK)]

        def x_start(ci, xb):
            pltpu.async_copy(x_slice(ci), xc_v.at[xb], xsems[xb])

        def x_wait(ci, xb):
            pltpu.make_async_copy(x_slice(ci), xc_v.at[xb], xsems[xb]).wait()

        def row_body(row, j, xb, skip_wait):
            p = j % _NOB
            if not skip_wait:
                pltpu.make_async_copy(
                    obuf_v.at[p], out_hbm.at[row - _NOB], osems[p]).wait()
            bits = [
                jnp.maximum(
                    1 - jnp.abs(xc_v[xb, j, pl.ds(o, 16)] - 1), 0)
                .astype(jnp.float32)
                for o in _OFFS
            ]

            def d_body(d, bits):
                r0 = r0s_v[pl.ds(d * 16, 16)]
                dr = drs_v[pl.ds(d * 16, 16)]
                for o, bv in zip(_OFFS, bits):
                    obuf_v[p, d, pl.ds(o, 16)] = bv * dr + r0
                return bits

            lax.fori_loop(0, _D, d_body, bits, unroll=False)
            pltpu.async_copy(obuf_v.at[p], out_hbm.at[row], osems[p])

        def chunk_body(ci, xb, first, guard_prefetch):
            row0 = base + ci * _XCHUNK
            x_wait(ci, xb)
            if guard_prefetch:
                @pl.when(ci + 1 < n_chunks)
                def _():
                    x_start(ci + 1, 1 - xb)
            else:
                x_start(ci + 1, 1 - xb)
            for j in range(_XCHUNK):
                row_body(row0 + j, j, xb, skip_wait=(first and j < _NOB))

        x_start(0, 0)
        chunk_body(0, 0, first=True, guard_prefetch=False)
        chunk_body(1, 1, first=False, guard_prefetch=False)

        def pair_body(kk, carry):
            chunk_body(2 * kk, 0, first=False, guard_prefetch=False)
            chunk_body(2 * kk + 1, 1, first=False, guard_prefetch=True)
            return carry

        lax.fori_loop(1, n_chunks // 2, pair_body, 0, unroll=False)

        last = base + rows_per_w
        for p in range(_NOB):
            pltpu.make_async_copy(
                obuf_v.at[p], out_hbm.at[last - _NOB + p], osems[p]).wait()

    return k


def kernel(x, table):
    t0 = table[0]
    r0s = jnp.repeat(t0, 16)
    drs = jnp.repeat(table[1] - t0, 16)
    return _build()(x, r0s, drs)


# speedup vs baseline: 4.4781x; 4.4781x over previous
import functools

import jax
import jax.numpy as jnp
from jax import lax
from jax.experimental import pallas as pl
from jax.experimental.pallas import tpu as pltpu
from jax.experimental.pallas import tpu_sc as plsc

_B, _L, _D = 16384, 200, 64
_BW = 512
_BT = 128
_NBT = _BW // _BT
_NOB = 2


@functools.lru_cache(maxsize=1)
def _build():
    info = plsc.get_sparse_core_info()
    nw = info.num_cores * info.num_subcores
    assert _B // nw == _BW

    mesh = plsc.VectorSubcoreMesh(core_axis_name="c", subcore_axis_name="s")

    @functools.partial(
        pl.kernel,
        out_type=jax.ShapeDtypeStruct((_D, _L, _B), jnp.float32),
        mesh=mesh,
        scratch_types=[
            pltpu.VMEM((_D * 16,), jnp.float32),
            pltpu.VMEM((_D * 16,), jnp.float32),
            pltpu.VMEM((_L, _BT), jnp.int32),
            pltpu.VMEM((_L, _BT), jnp.float32),
            pltpu.VMEM((_NOB, _L, _BT), jnp.float32),
            [pltpu.SemaphoreType.DMA] * _NOB,
        ],
    )
    def k(xt_hbm, r0s_hbm, drs_hbm, out_hbm,
          r0s_v, drs_v, xb_v, bit_v, obuf_v, osems):
        c = lax.axis_index("c")
        s = lax.axis_index("s")
        wid = s * info.num_cores + c
        bbase = pl.multiple_of(wid * _BW, _BW)

        pltpu.sync_copy(r0s_hbm, r0s_v)
        pltpu.sync_copy(drs_hbm, drs_v)

        dummy_dst = out_hbm.at[0, pl.ds(0, _L), pl.ds(0, _BT)]

        def blk_wait(q):
            pltpu.make_async_copy(obuf_v.at[q], dummy_dst, osems[q]).wait()

        def blk_emit(d_idx, bcol, q, skip_wait):
            if not skip_wait:
                blk_wait(q)
            r0 = r0s_v[pl.ds(d_idx * 16, 16)]
            dr = drs_v[pl.ds(d_idx * 16, 16)]

            def lp_body(lp, cc):
                for j in range(_BT // 16):
                    o = j * 16
                    obuf_v[q, lp, pl.ds(o, 16)] = (
                        bit_v[lp, pl.ds(o, 16)] * dr + r0)
                return cc

            lax.fori_loop(0, _L, lp_body, 0, unroll=False)
            pltpu.async_copy(
                obuf_v.at[q],
                out_hbm.at[d_idx, pl.ds(0, _L), pl.ds(bcol, _BT)],
                osems[q])

        for bt in range(_NBT):
            bcol = pl.multiple_of(bbase + bt * _BT, _BT)
            pltpu.sync_copy(xt_hbm.at[pl.ds(0, _L), pl.ds(bcol, _BT)], xb_v)

            def bin_body(lp, cc):
                for j in range(_BT // 16):
                    o = j * 16
                    bit_v[lp, pl.ds(o, 16)] = (
                        jnp.maximum(
                            1 - jnp.abs(xb_v[lp, pl.ds(o, 16)] - 1), 0)
                        .astype(jnp.float32))
                return cc

            lax.fori_loop(0, _L, bin_body, 0, unroll=False)

            def dd_body(dd, carry):
                d2 = dd * _NOB
                for q in range(_NOB):
                    blk_emit(d2 + q, bcol, q, skip_wait=False)
                return carry

            if bt == 0:
                for q in range(_NOB):
                    blk_emit(q, bcol, q, skip_wait=True)
                lax.fori_loop(1, _D // _NOB, dd_body, 0, unroll=False)
            else:
                lax.fori_loop(0, _D // _NOB, dd_body, 0, unroll=False)

        for q in range(_NOB):
            blk_wait(q)

    return k


def kernel(x, table):
    t0 = table[0]
    r0s = jnp.repeat(t0, 16)
    drs = jnp.repeat(table[1] - t0, 16)
    out3 = _build()(x.T, r0s, drs)
    return jnp.transpose(out3, (2, 0, 1))
